# Initial kernel scaffold; baseline (speedup 1.0000x reference)
import jax
import jax.numpy as jnp
from jax.experimental import pallas as pl


def kernel(In, NNsites, Weights, bias):
    raise NotImplementedError("write your pallas kernel here")


# speedup vs baseline: 24.0083x; 24.0083x over previous
import functools

import jax
import jax.numpy as jnp
from jax import lax
from jax.experimental import pallas as pl
from jax.experimental.pallas import tpu as pltpu
from jax.experimental.pallas import tpu_sc as plsc

_NC = 2
_NS = 16
_NW = _NC * _NS
_LANES = 16
_T = 32


def _linear_stage(In_p, Weights, bias, Npad, B, C_in, C_out, TB, interpret=False):
    K = B * C_out

    def body(in_ref, w_ref, b_ref, self_ref, p_ref):
        W = w_ref[0]
        bv = b_ref[0]
        for b in range(B):
            x = in_ref[b]
            sp = lax.dot_general(
                x, W[:, :C_in], (((0,), (1,)), ((), ())),
                preferred_element_type=jnp.float32,
                precision=lax.Precision.HIGHEST)
            npart = lax.dot_general(
                x, W[:, C_in:], (((0,), (1,)), ((), ())),
                preferred_element_type=jnp.float32,
                precision=lax.Precision.HIGHEST)
            self_ref[:, b * C_out:(b + 1) * C_out] = sp + bv[None, :]
            p_ref[:, b * C_out:(b + 1) * C_out] = npart

    grid = (Npad // TB,)
    return pl.pallas_call(
        body,
        grid=grid,
        in_specs=[
            pl.BlockSpec((B, C_in, TB), lambda i: (0, 0, i)),
            pl.BlockSpec(Weights.shape, lambda i: (0, 0, 0)),
            pl.BlockSpec(bias.shape, lambda i: (0, 0)),
        ],
        out_specs=[
            pl.BlockSpec((TB, K), lambda i: (i, 0)),
            pl.BlockSpec((TB, K), lambda i: (i, 0)),
        ],
        out_shape=[
            jax.ShapeDtypeStruct((Npad, K), jnp.float32),
            jax.ShapeDtypeStruct((Npad, K), jnp.float32),
        ],
        interpret=interpret,
    )(In_p, Weights, bias)


def _ln(p):
    bits = lax.bitcast_convert_type(p, jnp.int32)
    e = lax.shift_right_logical(bits, 23) - 127
    mbits = jnp.bitwise_or(jnp.bitwise_and(bits, 0x7FFFFF), 0x3F800000)
    m = lax.bitcast_convert_type(mbits, jnp.float32)
    big = m > 1.4142135623730951
    m = jnp.where(big, m * 0.5, m)
    ef = e.astype(jnp.float32) + jnp.where(big, 1.0, 0.0)
    r = (m - 1.0) / (m + 1.0)
    r2 = r * r
    poly = 1.0 + r2 * (0.3333333432674408 + r2 * (0.20000000298023224
                                                  + r2 * 0.14285714924335480))
    return ef * 0.6931471805599453 + (2.0 * r) * poly


def _gather_softplus_stage(nnt2, self_rows, p_rows, Npad, K, Z, per_w,
                           interpret=False):
    T = _T
    pairs = per_w // (2 * T)
    n_gath = (T * Z) // 128
    kv = K // _LANES

    mesh = plsc.VectorSubcoreMesh(core_axis_name="c", subcore_axis_name="s",
                                  num_cores=_NC, num_subcores=_NS)

    @functools.partial(
        pl.kernel,
        out_type=jax.ShapeDtypeStruct((Npad, K), jnp.float32),
        mesh=mesh,
        scratch_types=[
            pltpu.VMEM((2 * n_gath, 128), jnp.int32),
            pltpu.VMEM((T, K), jnp.float32),
            pltpu.VMEM((T * Z, K), jnp.float32),
            pltpu.VMEM((T, K), jnp.float32),
        ],
        compiler_params=pltpu.CompilerParams(use_tc_tiling_on_sc=False),
        interpret=interpret,
    )
    def run(nnt_hbm, self_hbm, p_hbm, out_hbm, idx_v, self_v, g_v, out_v):
        wid = lax.axis_index("s") * _NC + lax.axis_index("c")
        base0 = wid * per_w

        def pair_body(cp, _):
            rowb = pl.multiple_of((base0 + cp * 2 * T) * Z // 128, 8)
            pltpu.sync_copy(nnt_hbm.at[pl.ds(rowb, 2 * n_gath)], idx_v)
            for sub in range(2):
                base = pl.multiple_of(base0 + cp * 2 * T + sub * T, T)
                pltpu.sync_copy(self_hbm.at[pl.ds(base, T)], self_v)
                for q in range(n_gath):
                    pltpu.sync_copy(p_hbm.at[idx_v.at[sub * n_gath + q]],
                                    g_v.at[pl.ds(q * 128, 128)])

                def site_body(i, _):
                    for j in range(kv):
                        sl = pl.ds(_LANES * j, _LANES)
                        s = self_v[i, sl]
                        ssum = jnp.zeros((_LANES,), jnp.float32)
                        prod = jnp.ones((_LANES,), jnp.float32)
                        for z in range(Z):
                            gv = g_v[i * Z + z, sl]
                            x = s + gv
                            ssum = ssum + jnp.maximum(x, 0.0)
                            nax = jnp.minimum(x, -x)
                            prod = prod * (1.0 + jnp.exp(nax))
                        out_v[i, sl] = ssum + _ln(prod)
                    return 0

                lax.fori_loop(0, T, site_body, 0)
                pltpu.sync_copy(out_v, out_hbm.at[pl.ds(base, T)])
            return 0

        lax.fori_loop(0, pairs, pair_body, 0)

    return run(nnt2, self_rows, p_rows)


def kernel(In, NNsites, Weights, bias):
    B, C_in, N = In.shape
    C_out = Weights.shape[1]
    Z = NNsites.shape[0] - 1
    K = B * C_out

    per_w = -(-N // (_NW * _T)) * _T
    Npad = per_w * _NW

    In_p = jnp.pad(In, ((0, 0), (0, 0), (0, Npad - N)))
    nnt = jnp.pad(jnp.transpose(NNsites[1:1 + Z]), ((0, Npad - N), (0, 0)))
    nnt2 = nnt.reshape(Npad * Z // 128, 128)

    TB = 1024 if Npad % 1024 == 0 else _T
    self_rows, p_rows = _linear_stage(In_p, Weights, bias, Npad, B, C_in,
                                      C_out, TB)
    out_rows = _gather_softplus_stage(nnt2, self_rows, p_rows, Npad, K, Z,
                                      per_w)
    return jnp.transpose(out_rows[:N].reshape(N, B, C_out), (1, 2, 0))


# speedup vs baseline: 25.2791x; 1.0529x over previous
import functools

import jax
import jax.numpy as jnp
from jax import lax
from jax.experimental import pallas as pl
from jax.experimental.pallas import tpu as pltpu
from jax.experimental.pallas import tpu_sc as plsc

_NC = 2
_NS = 16
_NW = _NC * _NS
_LANES = 16
_T = 32


def _linear_stage(In_p, Weights, bias, Npad, B, C_in, C_out, TB, interpret=False):
    K = B * C_out

    def body(in_ref, w_ref, b_ref, self_ref, p_ref):
        W = w_ref[0]
        bv = b_ref[0]
        for b in range(B):
            x = in_ref[b]
            sp = lax.dot_general(
                x, W[:, :C_in], (((0,), (1,)), ((), ())),
                preferred_element_type=jnp.float32,
                precision=lax.Precision.HIGHEST)
            npart = lax.dot_general(
                x, W[:, C_in:], (((0,), (1,)), ((), ())),
                preferred_element_type=jnp.float32,
                precision=lax.Precision.HIGHEST)
            self_ref[:, b * C_out:(b + 1) * C_out] = sp + bv[None, :]
            p_ref[:, b * C_out:(b + 1) * C_out] = npart

    grid = (Npad // TB,)
    return pl.pallas_call(
        body,
        grid=grid,
        in_specs=[
            pl.BlockSpec((B, C_in, TB), lambda i: (0, 0, i)),
            pl.BlockSpec(Weights.shape, lambda i: (0, 0, 0)),
            pl.BlockSpec(bias.shape, lambda i: (0, 0)),
        ],
        out_specs=[
            pl.BlockSpec((TB, K), lambda i: (i, 0)),
            pl.BlockSpec((TB, K), lambda i: (i, 0)),
        ],
        out_shape=[
            jax.ShapeDtypeStruct((Npad, K), jnp.float32),
            jax.ShapeDtypeStruct((Npad, K), jnp.float32),
        ],
        interpret=interpret,
    )(In_p, Weights, bias)


def _ln(p):
    bits = lax.bitcast_convert_type(p, jnp.int32)
    e = lax.shift_right_logical(bits, 23) - 127
    mbits = jnp.bitwise_or(jnp.bitwise_and(bits, 0x7FFFFF), 0x3F800000)
    m = lax.bitcast_convert_type(mbits, jnp.float32)
    big = m > 1.4142135623730951
    m = jnp.where(big, m * 0.5, m)
    ef = e.astype(jnp.float32) + jnp.where(big, 1.0, 0.0)
    r = (m - 1.0) / (m + 1.0)
    r2 = r * r
    poly = 1.0 + r2 * (0.3333333432674408 + r2 * (0.20000000298023224
                                                  + r2 * 0.14285714924335480))
    return ef * 0.6931471805599453 + (2.0 * r) * poly


def _gather_softplus_stage(nnt2, self_rows, p_rows, Npad, K, Z, per_w,
                           interpret=False):
    T = _T
    n_gath = (T * Z) // 128
    kv = K // _LANES
    bodies = per_w // (4 * T)
    row0_step = 2 * T * Z // 128

    mesh = plsc.VectorSubcoreMesh(core_axis_name="c", subcore_axis_name="s",
                                  num_cores=_NC, num_subcores=_NS)

    @functools.partial(
        pl.kernel,
        out_type=jax.ShapeDtypeStruct((Npad, K), jnp.float32),
        mesh=mesh,
        scratch_types=[
            pltpu.VMEM((2 * n_gath, 128), jnp.int32),
            pltpu.VMEM((2 * n_gath, 128), jnp.int32),
            pltpu.VMEM((2, T, K), jnp.float32),
            pltpu.VMEM((2, T * Z, K), jnp.float32),
            pltpu.VMEM((2, T, K), jnp.float32),
            pltpu.SemaphoreType.DMA,
            pltpu.SemaphoreType.DMA,
            pltpu.SemaphoreType.DMA,
            pltpu.SemaphoreType.DMA,
            pltpu.SemaphoreType.DMA,
        ],
        compiler_params=pltpu.CompilerParams(use_tc_tiling_on_sc=False),
        interpret=interpret,
    )
    def run(nnt_hbm, self_hbm, p_hbm, out_hbm, idx_a, idx_b, self_v, g_v,
            out_v, sem_g0, sem_g1, sem_o0, sem_o1, sem_i):
        wid = lax.axis_index("s") * _NC + lax.axis_index("c")
        base0 = wid * per_w
        row0 = wid * (per_w * Z // 128)
        sem_g = (sem_g0, sem_g1)
        sem_o = (sem_o0, sem_o1)

        def issue_chunk(c, slot, idx_ref, half):
            base = pl.multiple_of(base0 + c * T, T)
            for q in range(n_gath):
                pltpu.async_copy(p_hbm.at[idx_ref.at[half * n_gath + q]],
                                 g_v.at[slot, pl.ds(q * 128, 128)],
                                 sem_g[slot])
            pltpu.async_copy(self_hbm.at[pl.ds(base, T)], self_v.at[slot],
                             sem_g[slot])

        def wait_chunk(slot):
            pltpu.make_async_copy(self_hbm.at[pl.ds(0, T * Z)],
                                  g_v.at[slot], sem_g[slot]).wait()
            pltpu.make_async_copy(self_hbm.at[pl.ds(0, T)],
                                  self_v.at[slot], sem_g[slot]).wait()

        def issue_idx(pr, idx_ref):
            rowb = pl.multiple_of(row0 + pr * row0_step, 8)
            pltpu.async_copy(nnt_hbm.at[pl.ds(rowb, 2 * n_gath)], idx_ref,
                             sem_i)

        def wait_idx(idx_ref):
            pltpu.make_async_copy(nnt_hbm.at[pl.ds(0, 2 * n_gath)], idx_ref,
                                  sem_i).wait()

        def store_out(c, slot):
            base = pl.multiple_of(base0 + c * T, T)
            pltpu.async_copy(out_v.at[slot], out_hbm.at[pl.ds(base, T)],
                             sem_o[slot])

        def wait_out(slot):
            pltpu.make_async_copy(out_v.at[slot], out_hbm.at[pl.ds(0, T)],
                                  sem_o[slot]).wait()

        def compute_chunk(slot):
            def site_body(i, _):
                for j in range(kv):
                    sl = pl.ds(_LANES * j, _LANES)
                    s = self_v[slot, i, sl]
                    ssum = jnp.zeros((_LANES,), jnp.float32)
                    prod = jnp.ones((_LANES,), jnp.float32)
                    for z in range(Z):
                        gv = g_v[slot, i * Z + z, sl]
                        x = s + gv
                        ssum = ssum + jnp.maximum(x, 0.0)
                        nax = jnp.minimum(x, -x)
                        prod = prod * (1.0 + jnp.exp(nax))
                    out_v[slot, i, sl] = ssum + _ln(prod)
                return 0

            lax.fori_loop(0, T, site_body, 0)

        pltpu.sync_copy(nnt_hbm.at[pl.ds(pl.multiple_of(row0, 8),
                                         2 * n_gath)], idx_a)
        issue_chunk(0, 0, idx_a, 0)

        def body(t, _):
            c0 = 4 * t
            issue_idx(2 * t + 1, idx_b)
            wait_chunk(0)
            issue_chunk(c0 + 1, 1, idx_a, 1)

            @pl.when(t > 0)
            def _():
                wait_out(0)
            compute_chunk(0)
            store_out(c0, 0)

            wait_idx(idx_b)
            wait_chunk(1)
            issue_chunk(c0 + 2, 0, idx_b, 0)

            @pl.when(t > 0)
            def _():
                wait_out(1)
            compute_chunk(1)
            store_out(c0 + 1, 1)

            @pl.when(t < bodies - 1)
            def _():
                issue_idx(2 * t + 2, idx_a)
            wait_chunk(0)
            issue_chunk(c0 + 3, 1, idx_b, 1)

            wait_out(0)
            compute_chunk(0)
            store_out(c0 + 2, 0)

            @pl.when(t < bodies - 1)
            def _():
                wait_idx(idx_a)
                issue_chunk(c0 + 4, 0, idx_a, 0)

            wait_chunk(1)
            wait_out(1)
            compute_chunk(1)
            store_out(c0 + 3, 1)
            return 0

        lax.fori_loop(0, bodies, body, 0)
        wait_out(0)
        wait_out(1)

    return run(nnt2, self_rows, p_rows)


def kernel(In, NNsites, Weights, bias):
    B, C_in, N = In.shape
    C_out = Weights.shape[1]
    Z = NNsites.shape[0] - 1
    K = B * C_out

    per_w = -(-N // (_NW * 4 * _T)) * 4 * _T
    Npad = per_w * _NW

    In_p = jnp.pad(In, ((0, 0), (0, 0), (0, Npad - N)))
    nnt = jnp.pad(jnp.transpose(NNsites[1:1 + Z]), ((0, Npad - N), (0, 0)))
    nnt2 = nnt.reshape(Npad * Z // 128, 128)

    TB = 1024 if Npad % 1024 == 0 else _T
    self_rows, p_rows = _linear_stage(In_p, Weights, bias, Npad, B, C_in,
                                      C_out, TB)
    out_rows = _gather_softplus_stage(nnt2, self_rows, p_rows, Npad, K, Z,
                                      per_w)
    return jnp.transpose(out_rows[:N].reshape(N, B, C_out), (1, 2, 0))


# speedup vs baseline: 29.1501x; 1.1531x over previous
import functools

import jax
import jax.numpy as jnp
from jax import lax
from jax.experimental import pallas as pl
from jax.experimental.pallas import tpu as pltpu
from jax.experimental.pallas import tpu_sc as plsc

_NC = 2
_NS = 16
_NW = _NC * _NS
_LANES = 16
_T = 32


def _linear_stage(In2, Wbig, bias_t, NN_p, Npad, K, Z, TB, interpret=False):
    def body(in_ref, w_ref, b_ref, nn_ref, self_ref, p_ref, nnt_ref):
        x = in_ref[...]
        y = lax.dot_general(x, w_ref[...], (((0,), (0,)), ((), ())),
                            preferred_element_type=jnp.float32,
                            precision=lax.Precision.HIGHEST)
        self_ref[...] = y[:, :K] + b_ref[0][None, :]
        p_ref[...] = y[:, K:]
        nnt_ref[...] = jnp.transpose(nn_ref[1:1 + Z, :])

    grid = (Npad // TB,)
    return pl.pallas_call(
        body,
        grid=grid,
        in_specs=[
            pl.BlockSpec((K, TB), lambda i: (0, i)),
            pl.BlockSpec(Wbig.shape, lambda i: (0, 0)),
            pl.BlockSpec(bias_t.shape, lambda i: (0, 0)),
            pl.BlockSpec((Z + 1, TB), lambda i: (0, i)),
        ],
        out_specs=[
            pl.BlockSpec((TB, K), lambda i: (i, 0)),
            pl.BlockSpec((TB, K), lambda i: (i, 0)),
            pl.BlockSpec((TB, Z), lambda i: (i, 0)),
        ],
        out_shape=[
            jax.ShapeDtypeStruct((Npad, K), jnp.float32),
            jax.ShapeDtypeStruct((Npad, K), jnp.float32),
            jax.ShapeDtypeStruct((Npad, Z), jnp.int32),
        ],
        interpret=interpret,
    )(In2, Wbig, bias_t, NN_p)


def _transpose_stage(rows, N, K, TB, interpret=False):
    def body(in_ref, out_ref):
        out_ref[...] = jnp.transpose(in_ref[...])

    grid = (-(-N // TB),)
    return pl.pallas_call(
        body,
        grid=grid,
        in_specs=[pl.BlockSpec((TB, K), lambda i: (i, 0))],
        out_specs=pl.BlockSpec((K, TB), lambda i: (0, i)),
        out_shape=jax.ShapeDtypeStruct((K, N), jnp.float32),
        interpret=interpret,
    )(rows)


def _ln(p):
    bits = lax.bitcast_convert_type(p, jnp.int32)
    e = lax.shift_right_logical(bits, 23) - 127
    mbits = jnp.bitwise_or(jnp.bitwise_and(bits, 0x7FFFFF), 0x3F800000)
    m = lax.bitcast_convert_type(mbits, jnp.float32)
    big = m > 1.4142135623730951
    m = jnp.where(big, m * 0.5, m)
    ef = e.astype(jnp.float32) + jnp.where(big, 1.0, 0.0)
    r = (m - 1.0) / (m + 1.0)
    r2 = r * r
    poly = 1.0 + r2 * (0.3333333432674408 + r2 * (0.20000000298023224
                                                  + r2 * 0.14285714924335480))
    return ef * 0.6931471805599453 + (2.0 * r) * poly


def _gather_softplus_stage(nnt2, self_rows, p_rows, Npad, K, Z, per_w,
                           interpret=False):
    T = _T
    n_gath = (T * Z) // 128
    kv = K // _LANES
    bodies = per_w // (4 * T)
    row0_step = 2 * T * Z // 128

    mesh = plsc.VectorSubcoreMesh(core_axis_name="c", subcore_axis_name="s",
                                  num_cores=_NC, num_subcores=_NS)

    @functools.partial(
        pl.kernel,
        out_type=jax.ShapeDtypeStruct((Npad, K), jnp.float32),
        mesh=mesh,
        scratch_types=[
            pltpu.VMEM((2 * n_gath, 128), jnp.int32),
            pltpu.VMEM((2 * n_gath, 128), jnp.int32),
            pltpu.VMEM((2, T, K), jnp.float32),
            pltpu.VMEM((2, T * Z, K), jnp.float32),
            pltpu.VMEM((2, T, K), jnp.float32),
            pltpu.SemaphoreType.DMA,
            pltpu.SemaphoreType.DMA,
            pltpu.SemaphoreType.DMA,
            pltpu.SemaphoreType.DMA,
            pltpu.SemaphoreType.DMA,
        ],
        compiler_params=pltpu.CompilerParams(use_tc_tiling_on_sc=False),
        interpret=interpret,
    )
    def run(nnt_hbm, self_hbm, p_hbm, out_hbm, idx_a, idx_b, self_v, g_v,
            out_v, sem_g0, sem_g1, sem_o0, sem_o1, sem_i):
        wid = lax.axis_index("s") * _NC + lax.axis_index("c")
        base0 = wid * per_w
        row0 = wid * (per_w * Z // 128)
        sem_g = (sem_g0, sem_g1)
        sem_o = (sem_o0, sem_o1)

        def issue_chunk(c, slot, idx_ref, half):
            base = pl.multiple_of(base0 + c * T, T)
            for q in range(n_gath):
                pltpu.async_copy(p_hbm.at[idx_ref.at[half * n_gath + q]],
                                 g_v.at[slot, pl.ds(q * 128, 128)],
                                 sem_g[slot])
            pltpu.async_copy(self_hbm.at[pl.ds(base, T)], self_v.at[slot],
                             sem_g[slot])

        def wait_chunk(slot):
            pltpu.make_async_copy(self_hbm.at[pl.ds(0, T * Z)],
                                  g_v.at[slot], sem_g[slot]).wait()
            pltpu.make_async_copy(self_hbm.at[pl.ds(0, T)],
                                  self_v.at[slot], sem_g[slot]).wait()

        def issue_idx(pr, idx_ref):
            rowb = pl.multiple_of(row0 + pr * row0_step, 8)
            pltpu.async_copy(nnt_hbm.at[pl.ds(rowb, 2 * n_gath)], idx_ref,
                             sem_i)

        def wait_idx(idx_ref):
            pltpu.make_async_copy(nnt_hbm.at[pl.ds(0, 2 * n_gath)], idx_ref,
                                  sem_i).wait()

        def store_out(c, slot):
            base = pl.multiple_of(base0 + c * T, T)
            pltpu.async_copy(out_v.at[slot], out_hbm.at[pl.ds(base, T)],
                             sem_o[slot])

        def wait_out(slot):
            pltpu.make_async_copy(out_v.at[slot], out_hbm.at[pl.ds(0, T)],
                                  sem_o[slot]).wait()

        def compute_chunk(slot):
            def site_body(i, _):
                for j in range(kv):
                    sl = pl.ds(_LANES * j, _LANES)
                    s = self_v[slot, i, sl]
                    ssum = jnp.zeros((_LANES,), jnp.float32)
                    prod = jnp.ones((_LANES,), jnp.float32)
                    for z in range(Z):
                        gv = g_v[slot, i * Z + z, sl]
                        x = s + gv
                        ssum = ssum + jnp.maximum(x, 0.0)
                        nax = jnp.minimum(x, -x)
                        prod = prod * (1.0 + jnp.exp(nax))
                    out_v[slot, i, sl] = ssum + _ln(prod)
                return 0

            lax.fori_loop(0, T, site_body, 0)

        pltpu.sync_copy(nnt_hbm.at[pl.ds(pl.multiple_of(row0, 8),
                                         2 * n_gath)], idx_a)
        issue_chunk(0, 0, idx_a, 0)

        def body(t, _):
            c0 = 4 * t
            issue_idx(2 * t + 1, idx_b)
            wait_chunk(0)
            issue_chunk(c0 + 1, 1, idx_a, 1)

            @pl.when(t > 0)
            def _():
                wait_out(0)
            compute_chunk(0)
            store_out(c0, 0)

            wait_idx(idx_b)
            wait_chunk(1)
            issue_chunk(c0 + 2, 0, idx_b, 0)

            @pl.when(t > 0)
            def _():
                wait_out(1)
            compute_chunk(1)
            store_out(c0 + 1, 1)

            @pl.when(t < bodies - 1)
            def _():
                issue_idx(2 * t + 2, idx_a)
            wait_chunk(0)
            issue_chunk(c0 + 3, 1, idx_b, 1)

            wait_out(0)
            compute_chunk(0)
            store_out(c0 + 2, 0)

            @pl.when(t < bodies - 1)
            def _():
                wait_idx(idx_a)
                issue_chunk(c0 + 4, 0, idx_a, 0)

            wait_chunk(1)
            wait_out(1)
            compute_chunk(1)
            store_out(c0 + 3, 1)
            return 0

        lax.fori_loop(0, bodies, body, 0)
        wait_out(0)
        wait_out(1)

    return run(nnt2, self_rows, p_rows)


def kernel(In, NNsites, Weights, bias):
    B, C_in, N = In.shape
    C_out = Weights.shape[1]
    Z = NNsites.shape[0] - 1
    K = B * C_out

    per_w = -(-N // (_NW * 4 * _T)) * 4 * _T
    Npad = per_w * _NW

    In2 = jnp.pad(In, ((0, 0), (0, 0), (0, Npad - N))).reshape(K, Npad)
    NN_p = jnp.pad(NNsites, ((0, 0), (0, Npad - N)))
    Wself = jnp.transpose(Weights[0, :, :C_in])
    Wnbr = jnp.transpose(Weights[0, :, C_in:])
    eye = jnp.eye(B, dtype=jnp.float32)
    Wbig = jnp.concatenate([jnp.kron(eye, Wself), jnp.kron(eye, Wnbr)],
                           axis=1)
    bias_t = jnp.tile(bias[0], B)[None, :]

    TB = 1024
    self_rows, p_rows, nnt = _linear_stage(In2, Wbig, bias_t, NN_p, Npad, K,
                                           Z, TB)
    nnt2 = nnt.reshape(Npad * Z // 128, 128)
    out_rows = _gather_softplus_stage(nnt2, self_rows, p_rows, Npad, K, Z,
                                      per_w)
    out_t = _transpose_stage(out_rows, N, K, TB)
    return out_t.reshape(B, C_out, N)


# speedup vs baseline: 30.5449x; 1.0479x over previous
import functools

import jax
import jax.numpy as jnp
from jax import lax
from jax.experimental import pallas as pl
from jax.experimental.pallas import tpu as pltpu
from jax.experimental.pallas import tpu_sc as plsc

_NC = 2
_NS = 16
_NW = _NC * _NS
_LANES = 16
_T = 32
_B0 = 31
_B1 = 18


def _linear_stage(In2, Wbig, bias_t, NN_p, Npad, K, Z, TB, interpret=False):
    def body(in_ref, w_ref, b_ref, nn_ref, self_ref, p_ref, nnt_ref):
        x = in_ref[...]
        y = lax.dot_general(x, w_ref[...], (((0,), (0,)), ((), ())),
                            preferred_element_type=jnp.float32,
                            precision=lax.Precision.HIGHEST)
        self_ref[...] = y[:, :K] + b_ref[0][None, :]
        p_ref[...] = y[:, K:]
        nnt_ref[...] = jnp.transpose(nn_ref[1:1 + Z, :])

    grid = (Npad // TB,)
    return pl.pallas_call(
        body,
        grid=grid,
        in_specs=[
            pl.BlockSpec((K, TB), lambda i: (0, i)),
            pl.BlockSpec(Wbig.shape, lambda i: (0, 0)),
            pl.BlockSpec(bias_t.shape, lambda i: (0, 0)),
            pl.BlockSpec((Z + 1, TB), lambda i: (0, i)),
        ],
        out_specs=[
            pl.BlockSpec((TB, K), lambda i: (i, 0)),
            pl.BlockSpec((TB, K), lambda i: (i, 0)),
            pl.BlockSpec((TB, Z), lambda i: (i, 0)),
        ],
        out_shape=[
            jax.ShapeDtypeStruct((Npad, K), jnp.float32),
            jax.ShapeDtypeStruct((Npad, K), jnp.float32),
            jax.ShapeDtypeStruct((Npad, Z), jnp.int32),
        ],
        interpret=interpret,
    )(In2, Wbig, bias_t, NN_p)


def _transpose_stage(rows, N, K, TB, interpret=False):
    def body(in_ref, out_ref):
        out_ref[...] = jnp.transpose(in_ref[...])

    grid = (-(-N // TB),)
    return pl.pallas_call(
        body,
        grid=grid,
        in_specs=[pl.BlockSpec((TB, K), lambda i: (i, 0))],
        out_specs=pl.BlockSpec((K, TB), lambda i: (0, i)),
        out_shape=jax.ShapeDtypeStruct((K, N), jnp.float32),
        interpret=interpret,
    )(rows)


def _ln(p):
    bits = lax.bitcast_convert_type(p, jnp.int32)
    e = lax.shift_right_logical(bits, 23) - 127
    mbits = jnp.bitwise_or(jnp.bitwise_and(bits, 0x7FFFFF), 0x3F800000)
    m = lax.bitcast_convert_type(mbits, jnp.float32)
    big = m > 1.4142135623730951
    m = jnp.where(big, m * 0.5, m)
    ef = e.astype(jnp.float32) + jnp.where(big, 1.0, 0.0)
    r = (m - 1.0) / (m + 1.0)
    r2 = r * r
    poly = 1.0 + r2 * (0.3333333432674408 + r2 * (0.20000000298023224
                                                  + r2 * 0.14285714924335480))
    return ef * 0.6931471805599453 + (2.0 * r) * poly


def _gather_softplus_stage(nnt2, self_rows, p_rows, Npad, K, Z,
                           interpret=False):
    T = _T
    n_gath = (T * Z) // 128
    kv = K // _LANES
    row0_step = 2 * T * Z // 128

    mesh = plsc.VectorSubcoreMesh(core_axis_name="c", subcore_axis_name="s",
                                  num_cores=_NC, num_subcores=_NS)

    @functools.partial(
        pl.kernel,
        out_type=jax.ShapeDtypeStruct((Npad, K), jnp.float32),
        mesh=mesh,
        scratch_types=[
            pltpu.VMEM((2 * n_gath, 128), jnp.int32),
            pltpu.VMEM((2 * n_gath, 128), jnp.int32),
            pltpu.VMEM((2, T, K), jnp.float32),
            pltpu.VMEM((2, T * Z, K), jnp.float32),
            pltpu.VMEM((2, T, K), jnp.float32),
            pltpu.SemaphoreType.DMA,
            pltpu.SemaphoreType.DMA,
            pltpu.SemaphoreType.DMA,
            pltpu.SemaphoreType.DMA,
            pltpu.SemaphoreType.DMA,
        ],
        compiler_params=pltpu.CompilerParams(use_tc_tiling_on_sc=False),
        interpret=interpret,
    )
    def run(nnt_hbm, self_hbm, p_hbm, out_hbm, idx_a, idx_b, self_v, g_v,
            out_v, sem_g0, sem_g1, sem_o0, sem_o1, sem_i):
        cid = lax.axis_index("c")
        sid = lax.axis_index("s")
        bodies = jnp.where(cid == 0, _B0, _B1)
        base0 = jnp.where(cid == 0, sid * _B0,
                          _NS * _B0 + sid * _B1) * (4 * T)
        row0 = base0 * Z // 128
        sem_g = (sem_g0, sem_g1)
        sem_o = (sem_o0, sem_o1)

        def issue_chunk(c, slot, idx_ref, half):
            base = pl.multiple_of(base0 + c * T, T)
            for q in range(n_gath):
                pltpu.async_copy(p_hbm.at[idx_ref.at[half * n_gath + q]],
                                 g_v.at[slot, pl.ds(q * 128, 128)],
                                 sem_g[slot])
            pltpu.async_copy(self_hbm.at[pl.ds(base, T)], self_v.at[slot],
                             sem_g[slot])

        def wait_chunk(slot):
            pltpu.make_async_copy(self_hbm.at[pl.ds(0, T * Z)],
                                  g_v.at[slot], sem_g[slot]).wait()
            pltpu.make_async_copy(self_hbm.at[pl.ds(0, T)],
                                  self_v.at[slot], sem_g[slot]).wait()

        def issue_idx(pr, idx_ref):
            rowb = pl.multiple_of(row0 + pr * row0_step, 8)
            pltpu.async_copy(nnt_hbm.at[pl.ds(rowb, 2 * n_gath)], idx_ref,
                             sem_i)

        def wait_idx(idx_ref):
            pltpu.make_async_copy(nnt_hbm.at[pl.ds(0, 2 * n_gath)], idx_ref,
                                  sem_i).wait()

        def store_out(c, slot):
            base = pl.multiple_of(base0 + c * T, T)
            pltpu.async_copy(out_v.at[slot], out_hbm.at[pl.ds(base, T)],
                             sem_o[slot])

        def wait_out(slot):
            pltpu.make_async_copy(out_v.at[slot], out_hbm.at[pl.ds(0, T)],
                                  sem_o[slot]).wait()

        def compute_chunk(slot):
            def site_body(i, _):
                for j in range(kv):
                    sl = pl.ds(_LANES * j, _LANES)
                    s = self_v[slot, i, sl]
                    ssum = jnp.zeros((_LANES,), jnp.float32)
                    prod = jnp.ones((_LANES,), jnp.float32)
                    for z in range(Z):
                        gv = g_v[slot, i * Z + z, sl]
                        x = s + gv
                        m = jnp.maximum(x, 0.0)
                        ssum = ssum + m
                        e = jnp.exp(x - 2.0 * m)
                        prod = prod * e + prod
                    out_v[slot, i, sl] = ssum + _ln(prod)
                return 0

            lax.fori_loop(0, T, site_body, 0)

        pltpu.sync_copy(nnt_hbm.at[pl.ds(pl.multiple_of(row0, 8),
                                         2 * n_gath)], idx_a)
        issue_chunk(0, 0, idx_a, 0)

        def body(t, _):
            c0 = 4 * t
            issue_idx(2 * t + 1, idx_b)
            wait_chunk(0)
            issue_chunk(c0 + 1, 1, idx_a, 1)

            @pl.when(t > 0)
            def _():
                wait_out(0)
            compute_chunk(0)
            store_out(c0, 0)

            wait_idx(idx_b)
            wait_chunk(1)
            issue_chunk(c0 + 2, 0, idx_b, 0)

            @pl.when(t > 0)
            def _():
                wait_out(1)
            compute_chunk(1)
            store_out(c0 + 1, 1)

            @pl.when(t < bodies - 1)
            def _():
                issue_idx(2 * t + 2, idx_a)
            wait_chunk(0)
            issue_chunk(c0 + 3, 1, idx_b, 1)

            wait_out(0)
            compute_chunk(0)
            store_out(c0 + 2, 0)

            @pl.when(t < bodies - 1)
            def _():
                wait_idx(idx_a)
                issue_chunk(c0 + 4, 0, idx_a, 0)

            wait_chunk(1)
            wait_out(1)
            compute_chunk(1)
            store_out(c0 + 3, 1)
            return 0

        lax.fori_loop(0, bodies, body, 0)
        wait_out(0)
        wait_out(1)

    return run(nnt2, self_rows, p_rows)


def kernel(In, NNsites, Weights, bias):
    B, C_in, N = In.shape
    C_out = Weights.shape[1]
    Z = NNsites.shape[0] - 1
    K = B * C_out

    Npad = _NS * (_B0 + _B1) * 4 * _T
    assert Npad >= N

    In2 = jnp.pad(In, ((0, 0), (0, 0), (0, Npad - N))).reshape(K, Npad)
    NN_p = jnp.pad(NNsites, ((0, 0), (0, Npad - N)))
    Wself = jnp.transpose(Weights[0, :, :C_in])
    Wnbr = jnp.transpose(Weights[0, :, C_in:])
    eye = jnp.eye(B, dtype=jnp.float32)
    Wbig = jnp.concatenate([jnp.kron(eye, Wself), jnp.kron(eye, Wnbr)],
                           axis=1)
    bias_t = jnp.tile(bias[0], B)[None, :]

    TB = 1024
    self_rows, p_rows, nnt = _linear_stage(In2, Wbig, bias_t, NN_p, Npad, K,
                                           Z, TB)
    nnt2 = nnt.reshape(Npad * Z // 128, 128)
    out_rows = _gather_softplus_stage(nnt2, self_rows, p_rows, Npad, K, Z)
    out_t = _transpose_stage(out_rows, N, K, TB)
    return out_t.reshape(B, C_out, N)
